# baseline (speedup 1.0000x reference)
import functools
import jax
import jax.numpy as jnp
from jax import lax
from jax.experimental import pallas as pl
from jax.experimental.pallas import tpu as pltpu
from jax.experimental.pallas import tpu_sc as plsc

_N = 131072
_BINS = 15
_BP = 16
_NO = 256
_NW = 32
_PW = _N // _NW
_CH = 64
_NCH = _PW // _CH
_NPAIR = _NCH // 2
_G = _CH // 16

_mesh = plsc.VectorSubcoreMesh(core_axis_name="c", subcore_axis_name="s")


@functools.partial(
    pl.kernel,
    out_type=jax.ShapeDtypeStruct((_N, _NO), jnp.float32),
    mesh=_mesh,
    compiler_params=pltpu.CompilerParams(use_tc_tiling_on_sc=False, needs_layout_passes=False),
    scratch_types=[
        pltpu.VMEM((_PW,), jnp.float32),
        pltpu.VMEM((_PW,), jnp.float32),
        pltpu.VMEM((32,), jnp.float32),
        pltpu.VMEM((_CH, _NO), jnp.float32),
        pltpu.VMEM((_CH, _NO), jnp.float32),
        pltpu.VMEM((_CH,), jnp.int32),
        pltpu.VMEM((_CH,), jnp.int32),
        pltpu.SemaphoreType.DMA,
        pltpu.SemaphoreType.DMA,
    ],
)
def _sc_tile_code(
    x0_hbm, x1_hbm, tiles_hbm, zeros_hbm, out_hbm,
    x0_v, x1_v, tiles_v, bufA, bufB, codesA, codesB, semA, semB,
):
    wid = lax.axis_index("s") * 2 + lax.axis_index("c")
    base = wid * _PW
    pltpu.sync_copy(x0_hbm.at[pl.ds(base, _PW)], x0_v)
    pltpu.sync_copy(x1_hbm.at[pl.ds(base, _PW)], x1_v)
    pltpu.sync_copy(tiles_hbm, tiles_v)
    pltpu.sync_copy(zeros_hbm, bufA)
    pltpu.sync_copy(zeros_hbm, bufB)
    ta = tiles_v[pl.ds(0, 16)]
    tb = tiles_v[pl.ds(16, 16)]
    t0 = [ta[b] for b in range(_BINS)]
    t1 = [tb[b] for b in range(_BINS)]
    lanes = lax.iota(jnp.int32, 16)
    ones_f = jnp.full((16,), 1.0, jnp.float32)
    zeros_f = jnp.zeros((16,), jnp.float32)
    zeros_i = jnp.zeros((16,), jnp.int32)
    for g in range(_G):
        codesA[pl.ds(g * 16, 16)] = zeros_i
        codesB[pl.ds(g * 16, 16)] = zeros_i

    def half(pair, buf, codes, sem, ch):
        row0 = base + ch * _CH
        dst = out_hbm.at[pl.ds(row0, _CH), :]

        @pl.when(pair > 0)
        def _():
            pltpu.make_async_copy(buf, dst, sem).wait()

        for g in range(_G):
            rows = lanes + (g * 16)
            prev = codes[pl.ds(g * 16, 16)]
            plsc.store_scatter(buf, [rows, prev], zeros_f)
        for g in range(_G):
            off = ch * _CH + g * 16
            xv0 = x0_v[pl.ds(off, 16)]
            xv1 = x1_v[pl.ds(off, 16)]
            c0 = jnp.zeros((16,), jnp.int32)
            c1 = jnp.zeros((16,), jnp.int32)
            for b in range(_BINS):
                c0 = c0 + (xv0 > t0[b]).astype(jnp.int32)
                c1 = c1 + (xv1 > t1[b]).astype(jnp.int32)
            code = c0 * _BP + c1
            rows = lanes + (g * 16)
            plsc.store_scatter(buf, [rows, code], ones_f)
            codes[pl.ds(g * 16, 16)] = code
        pltpu.make_async_copy(buf, dst, sem).start()

    def body(pair, carry):
        half(pair, bufA, codesA, semA, 2 * pair)
        half(pair, bufB, codesB, semB, 2 * pair + 1)
        return carry

    lax.fori_loop(0, _NPAIR, body, 0)
    pltpu.make_async_copy(
        bufA, out_hbm.at[pl.ds(base + (_NCH - 2) * _CH, _CH), :], semA
    ).wait()
    pltpu.make_async_copy(
        bufB, out_hbm.at[pl.ds(base + (_NCH - 1) * _CH, _CH), :], semB
    ).wait()


def kernel(x, tiles):
    x0 = x[:, 0] + 0.0
    x1 = x[:, 1] + 0.0
    tiles_pad = jnp.concatenate(
        [tiles[:, 0], jnp.zeros((1,), jnp.float32),
         tiles[:, 1], jnp.zeros((1,), jnp.float32)]
    )
    zeros = jnp.zeros((_CH, _NO), jnp.float32)
    return _sc_tile_code(x0, x1, tiles_pad, zeros)


# speedup vs baseline: 1.2184x; 1.2184x over previous
import functools
import jax
import jax.numpy as jnp
from jax import lax
from jax.experimental import pallas as pl
from jax.experimental.pallas import tpu as pltpu
from jax.experimental.pallas import tpu_sc as plsc

_N = 131072
_DIM = 2
_BINS = 15
_BP = 16
_NO = 256

_NSC = 24576
_NTC = _N - _NSC
_BLK = 8192
_SUB = _BLK // 128

_NW = 32
_PW = _NSC // _NW
_CH = 128
_NCH = _PW // _CH
_NPAIR = _NCH // 2
_G = _CH // 16


def _tile_code_block(x0_ref, x1_ref, tiles_ref, out_ref):
    x0 = x0_ref[...]
    x1 = x1_ref[...]
    cnt0 = jnp.zeros(x0.shape, jnp.int32)
    cnt1 = jnp.zeros(x1.shape, jnp.int32)
    for b in range(_BINS):
        cnt0 = cnt0 + (x0 > tiles_ref[b : b + 1, 0:1]).astype(jnp.int32)
        cnt1 = cnt1 + (x1 > tiles_ref[b : b + 1, 1:2]).astype(jnp.int32)
    code = _BP * cnt0 + cnt1
    cols = jax.lax.broadcasted_iota(jnp.int32, (_SUB, _NO), 1)
    for k in range(128):
        col = code[:, k : k + 1]
        out_ref[_SUB * k : _SUB * (k + 1), :] = (cols == col).astype(
            jnp.float32
        )


def _tc_part(x, tiles):
    nb = _NTC // _BLK
    x0g = (
        x[:_NTC, 0].reshape(nb, 128, _SUB).transpose(0, 2, 1).reshape(nb * _SUB, 128)
    )
    x1g = (
        x[:_NTC, 1].reshape(nb, 128, _SUB).transpose(0, 2, 1).reshape(nb * _SUB, 128)
    )
    return pl.pallas_call(
        _tile_code_block,
        grid=(nb,),
        in_specs=[
            pl.BlockSpec((_SUB, 128), lambda i: (i, 0)),
            pl.BlockSpec((_SUB, 128), lambda i: (i, 0)),
            pl.BlockSpec((_BINS, _DIM), lambda i: (0, 0)),
        ],
        out_specs=pl.BlockSpec((_BLK, _NO), lambda i: (i, 0)),
        out_shape=jax.ShapeDtypeStruct((_NTC, _NO), jnp.float32),
    )(x0g, x1g, tiles)


_mesh = plsc.VectorSubcoreMesh(core_axis_name="c", subcore_axis_name="s")


@functools.partial(
    pl.kernel,
    out_type=jax.ShapeDtypeStruct((_NSC, _NO), jnp.float32),
    mesh=_mesh,
    compiler_params=pltpu.CompilerParams(
        use_tc_tiling_on_sc=False, needs_layout_passes=False
    ),
    scratch_types=[
        pltpu.VMEM((_PW,), jnp.float32),
        pltpu.VMEM((_PW,), jnp.float32),
        pltpu.VMEM((32,), jnp.float32),
        pltpu.VMEM((_CH, _NO), jnp.float32),
        pltpu.VMEM((_CH, _NO), jnp.float32),
        pltpu.VMEM((_CH,), jnp.int32),
        pltpu.VMEM((_CH,), jnp.int32),
        pltpu.SemaphoreType.DMA,
        pltpu.SemaphoreType.DMA,
    ],
)
def _sc_tile_code(
    x0_hbm, x1_hbm, tiles_hbm, zeros_hbm, out_hbm,
    x0_v, x1_v, tiles_v, bufA, bufB, codesA, codesB, semA, semB,
):
    wid = lax.axis_index("s") * 2 + lax.axis_index("c")
    base = wid * _PW
    pltpu.sync_copy(x0_hbm.at[pl.ds(base, _PW)], x0_v)
    pltpu.sync_copy(x1_hbm.at[pl.ds(base, _PW)], x1_v)
    pltpu.sync_copy(tiles_hbm, tiles_v)
    pltpu.sync_copy(zeros_hbm, bufA)
    pltpu.sync_copy(zeros_hbm, bufB)
    ta = tiles_v[pl.ds(0, 16)]
    tb = tiles_v[pl.ds(16, 16)]
    t0 = [ta[b] for b in range(_BINS)]
    t1 = [tb[b] for b in range(_BINS)]
    lanes = lax.iota(jnp.int32, 16)
    ones_f = jnp.full((16,), 1.0, jnp.float32)
    zeros_f = jnp.zeros((16,), jnp.float32)
    zeros_i = jnp.zeros((16,), jnp.int32)
    for g in range(_G):
        codesA[pl.ds(g * 16, 16)] = zeros_i
        codesB[pl.ds(g * 16, 16)] = zeros_i

    def half(pair, buf, codes, sem, ch):
        row0 = base + ch * _CH
        dst = out_hbm.at[pl.ds(row0, _CH), :]

        @pl.when(pair > 0)
        def _():
            pltpu.make_async_copy(buf, dst, sem).wait()

        for g in range(_G):
            rows = lanes + (g * 16)
            prev = codes[pl.ds(g * 16, 16)]
            plsc.store_scatter(buf, [rows, prev], zeros_f)
        for g in range(_G):
            off = ch * _CH + g * 16
            xv0 = x0_v[pl.ds(off, 16)]
            xv1 = x1_v[pl.ds(off, 16)]
            c0 = jnp.zeros((16,), jnp.int32)
            c1 = jnp.zeros((16,), jnp.int32)
            for b in range(_BINS):
                c0 = c0 + (xv0 > t0[b]).astype(jnp.int32)
                c1 = c1 + (xv1 > t1[b]).astype(jnp.int32)
            code = c0 * _BP + c1
            rows = lanes + (g * 16)
            plsc.store_scatter(buf, [rows, code], ones_f)
            codes[pl.ds(g * 16, 16)] = code
        pltpu.make_async_copy(buf, dst, sem).start()

    def body(pair, carry):
        half(pair, bufA, codesA, semA, 2 * pair)
        half(pair, bufB, codesB, semB, 2 * pair + 1)
        return carry

    lax.fori_loop(0, _NPAIR, body, 0)
    pltpu.make_async_copy(
        bufA, out_hbm.at[pl.ds(base + (_NCH - 2) * _CH, _CH), :], semA
    ).wait()
    pltpu.make_async_copy(
        bufB, out_hbm.at[pl.ds(base + (_NCH - 1) * _CH, _CH), :], semB
    ).wait()


def kernel(x, tiles):
    tc_out = _tc_part(x, tiles)
    x0 = x[_NTC:, 0] + 0.0
    x1 = x[_NTC:, 1] + 0.0
    tiles_flat = jnp.concatenate(
        [tiles[:, 0], jnp.zeros((1,), jnp.float32),
         tiles[:, 1], jnp.zeros((1,), jnp.float32)]
    )
    zeros = jnp.zeros((_CH, _NO), jnp.float32)
    sc_out = _sc_tile_code(x0, x1, tiles_flat, zeros)
    return jnp.concatenate([tc_out, sc_out], axis=0)


# speedup vs baseline: 4.3961x; 3.6080x over previous
import jax
import jax.numpy as jnp
from jax.experimental import pallas as pl

_N = 131072
_DIM = 2
_BINS = 15
_BP = _BINS + 1
_NUM_OUTPUTS = _BP * _BP
_BLK = 16384
_SUB = _BLK // 128


def _tile_code_block(x0_ref, x1_ref, tiles_ref, out_ref):
    x0 = x0_ref[...]
    x1 = x1_ref[...]
    cnt0 = jnp.zeros(x0.shape, jnp.int32)
    cnt1 = jnp.zeros(x1.shape, jnp.int32)
    for b in range(_BINS):
        cnt0 = cnt0 + (x0 > tiles_ref[b : b + 1, 0:1]).astype(jnp.int32)
        cnt1 = cnt1 + (x1 > tiles_ref[b : b + 1, 1:2]).astype(jnp.int32)
    code = _BP * cnt0 + cnt1
    cols = jax.lax.broadcasted_iota(jnp.int32, (_SUB, _NUM_OUTPUTS), 1)
    for k in range(128):
        col = code[:, k : k + 1]
        out_ref[_SUB * k : _SUB * (k + 1), :] = (cols == col).astype(
            jnp.float32
        )


def kernel(x, tiles):
    nb = _N // _BLK
    x0g = x[:, 0].reshape(nb, 128, _SUB).transpose(0, 2, 1).reshape(nb * _SUB, 128)
    x1g = x[:, 1].reshape(nb, 128, _SUB).transpose(0, 2, 1).reshape(nb * _SUB, 128)
    return pl.pallas_call(
        _tile_code_block,
        grid=(nb,),
        in_specs=[
            pl.BlockSpec((_SUB, 128), lambda i: (i, 0)),
            pl.BlockSpec((_SUB, 128), lambda i: (i, 0)),
            pl.BlockSpec((_BINS, _DIM), lambda i: (0, 0)),
        ],
        out_specs=pl.BlockSpec((_BLK, _NUM_OUTPUTS), lambda i: (i, 0)),
        out_shape=jax.ShapeDtypeStruct((_N, _NUM_OUTPUTS), jnp.float32),
    )(x0g, x1g, tiles)
